# Initial kernel scaffold; baseline (speedup 1.0000x reference)
import jax
import jax.numpy as jnp
from jax.experimental import pallas as pl


def kernel(x, edge_index, W1, b1, W2, b2, Wg1, asrc1, adst1, bg1, Wg2, asrc2, adst2, bg2, Wg3, asrc3, adst3, bg3):
    raise NotImplementedError("write your pallas kernel here")


# speedup vs baseline: 1.1212x; 1.1212x over previous
import functools
import jax
import jax.numpy as jnp
from jax.experimental import pallas as pl
from jax.experimental.pallas import tpu as pltpu

_HEADS = 8
_N = 10000
_E = 320000


def _mm_kernel(x_ref, w_ref, b_ref, o_ref):
    o_ref[...] = jnp.dot(x_ref[...], w_ref[...],
                         preferred_element_type=jnp.float32) + b_ref[...]


def _matmul_bias(x, w, b, block_rows=2000):
    n, k = x.shape
    f = w.shape[1]
    grid = (n // block_rows,)
    return pl.pallas_call(
        _mm_kernel,
        grid=grid,
        in_specs=[
            pl.BlockSpec((block_rows, k), lambda i: (i, 0)),
            pl.BlockSpec((k, f), lambda i: (0, 0)),
            pl.BlockSpec((f,), lambda i: (0,)),
        ],
        out_specs=pl.BlockSpec((block_rows, f), lambda i: (i, 0)),
        out_shape=jax.ShapeDtypeStruct((n, f), jnp.float32),
    )(x, w, b)


def _gcn(x, src, dst, dinv, W, b, n):
    xw = _matmul_bias(x, W, jnp.zeros((W.shape[1],), jnp.float32))
    y = xw * dinv[:, None]
    agg = jax.ops.segment_sum(y[src], dst, num_segments=n)
    return agg * dinv[:, None] + b


def _gat(x, src, dst, W, a_src, a_dst, b, heads, out_ch, concat, n):
    xw = _matmul_bias(x, W, jnp.zeros((W.shape[1],), jnp.float32))
    xwh = xw.reshape(n, heads, out_ch)
    al_s = jnp.sum(xwh * a_src, axis=-1)
    al_d = jnp.sum(xwh * a_dst, axis=-1)
    amax = jnp.max(al_s, axis=0)
    m = jax.nn.leaky_relu(al_d + amax[None, :], negative_slope=0.2)
    e = jax.nn.leaky_relu(al_s[src] + al_d[dst], negative_slope=0.2)
    ee = jnp.exp(e - m[dst])
    denom = jax.ops.segment_sum(ee, dst, num_segments=n)
    msg = xwh[src] * ee[:, :, None]
    out = jax.ops.segment_sum(msg, dst, num_segments=n)
    out = out / (denom[:, :, None] + 1e-16)
    if concat:
        out = out.reshape(n, heads * out_ch)
    else:
        out = out.mean(axis=1)
    return out + b


def kernel(x, edge_index, W1, b1, W2, b2, Wg1, asrc1, adst1, bg1, Wg2,
           asrc2, adst2, bg2, Wg3, asrc3, adst3, bg3):
    n = x.shape[0]
    loop = jnp.arange(n, dtype=edge_index.dtype)
    src = jnp.concatenate([edge_index[0], loop])
    dst = jnp.concatenate([edge_index[1], loop])

    deg = jax.ops.segment_sum(jnp.ones_like(src, dtype=jnp.float32), dst,
                              num_segments=n)
    dinv = jnp.where(deg > 0, 1.0 / jnp.sqrt(jnp.where(deg > 0, deg, 1.0)),
                     0.0)

    h = jax.nn.elu(_gcn(x, src, dst, dinv, W1, b1, n))
    h = jax.nn.elu(_gcn(h, src, dst, dinv, W2, b2, n))
    h = jax.nn.elu(_gat(h, src, dst, Wg1, asrc1, adst1, bg1, _HEADS, 64,
                        True, n))
    h = jax.nn.elu(_gat(h, src, dst, Wg2, asrc2, adst2, bg2, _HEADS, 64,
                        True, n))
    h = _gat(h, src, dst, Wg3, asrc3, adst3, bg3, 1, 64, False, n)
    return jax.nn.log_softmax(h, axis=1)
